# baseline (speedup 1.0000x reference)
import functools

import jax
import jax.numpy as jnp
from jax import lax
from jax.experimental import pallas as pl
from jax.experimental.pallas import tpu as pltpu
from jax.experimental.pallas import tpu_sc as plsc

VOCAB = 1000000
MAX_LEN = 200
EMBED = 32
B = 16384
L = 50
N = B * L

NUM_CORES = 2
NUM_SUBCORES = 16
NW = NUM_CORES * NUM_SUBCORES
ROWS_W = N // NW
CHUNK = 128
NCHUNK = ROWS_W // CHUNK
NBUF = 2
NITER = NCHUNK // NBUF
TB = VOCAB * EMBED // 128

_mesh = plsc.VectorSubcoreMesh(core_axis_name="c", subcore_axis_name="s")


@functools.partial(
    pl.kernel,
    out_type=jax.ShapeDtypeStruct((N, EMBED), jnp.float32),
    mesh=_mesh,
    compiler_params=pltpu.CompilerParams(needs_layout_passes=False,
                                         use_tc_tiling_on_sc=True),
    scratch_types=[
        pltpu.VMEM((NBUF, CHUNK), jnp.int32),
        pltpu.VMEM((NBUF, CHUNK), jnp.int32),
        pltpu.VMEM((NBUF, CHUNK), jnp.int32),
        pltpu.VMEM((NBUF, CHUNK), jnp.int32),
        pltpu.VMEM((NBUF, CHUNK, 128), jnp.float32),
        pltpu.VMEM((NBUF, CHUNK, EMBED), jnp.float32),
        pltpu.VMEM((MAX_LEN * EMBED,), jnp.float32),
        pltpu.VMEM((MAX_LEN * EMBED,), jnp.float32),
        pltpu.VMEM((MAX_LEN * EMBED,), jnp.float32),
        [pltpu.SemaphoreType.DMA] * NBUF,
        [pltpu.SemaphoreType.DMA] * NBUF,
    ],
)
def _packet_embed(tok, fld, hdr, temb, tpe, fpe, hpe, out,
                  tok_v, blk_v, fld_v, hdr_v, gbuf_v, obuf_v,
                  tpe_v, fpe_v, hpe_v, gsems, osems):
    wid = lax.axis_index("s") * NUM_CORES + lax.axis_index("c")
    base_w = wid * ROWS_W

    pltpu.sync_copy(tpe, tpe_v)
    pltpu.sync_copy(fpe, fpe_v)
    pltpu.sync_copy(hpe, hpe_v)

    def stage(c, s):
        base = base_w + c * CHUNK
        pltpu.sync_copy(tok.at[pl.ds(base, CHUNK)], tok_v.at[s])
        pltpu.sync_copy(fld.at[pl.ds(base, CHUNK)], fld_v.at[s])
        pltpu.sync_copy(hdr.at[pl.ds(base, CHUNK)], hdr_v.at[s])
        for g in range(CHUNK // 16):
            r0 = g * 16
            blk_v[s, pl.ds(r0, 16)] = lax.shift_right_logical(
                tok_v[s, pl.ds(r0, 16)], 2)
        pltpu.async_copy(temb.at[blk_v.at[s]], gbuf_v.at[s], gsems[s])

    def drain_gather(s):
        pltpu.make_async_copy(temb.at[blk_v.at[s]], gbuf_v.at[s],
                              gsems[s]).wait()

    def fire_out(c, s):
        base = base_w + c * CHUNK
        pltpu.async_copy(obuf_v.at[s], out.at[pl.ds(base, CHUNK)], osems[s])

    def wait_out(c, s):
        base = base_w + c * CHUNK
        pltpu.make_async_copy(obuf_v.at[s], out.at[pl.ds(base, CHUNK)],
                              osems[s]).wait()

    def compute(c, s):
        base = base_w + c * CHUNK

        def group_body(g, inner):
            r0 = g * 16
            qvec = lax.bitwise_and(tok_v[s, pl.ds(r0, 16)], 3)
            fvec = fld_v[s, pl.ds(r0, 16)]
            hvec = hdr_v[s, pl.ds(r0, 16)]
            for k in range(16):
                r = r0 + k
                q32 = qvec[k] * 32
                f32o = fvec[k] * 32
                h32o = hvec[k] * 32
                l32 = lax.rem(base + r, L) * 32
                for half in (0, 16):
                    acc = (gbuf_v[s, r, pl.ds(q32 + half, 16)]
                           + tpe_v[pl.ds(l32 + half, 16)]
                           + fpe_v[pl.ds(f32o + half, 16)]
                           + hpe_v[pl.ds(h32o + half, 16)])
                    obuf_v[s, r, pl.ds(half, 16)] = acc
            return inner

        lax.fori_loop(0, CHUNK // 16, group_body, 0)

    stage(0, 0)

    def iter_body(k, carry):
        ii = k * NBUF
        for j in range(NBUF):
            c = ii + j
            sn = (j + 1) % NBUF

            @pl.when(jnp.logical_and(c >= NBUF - 1, c + 1 < NCHUNK))
            def _():
                wait_out(c + 1 - NBUF, sn)

            @pl.when(c + 1 < NCHUNK)
            def _():
                stage(c + 1, sn)

            drain_gather(j)
            compute(c, j)
            fire_out(c, j)
        return carry

    lax.fori_loop(0, NITER, iter_body, 0)

    for j in range(NBUF):
        wait_out(NCHUNK - NBUF + j, j)


def kernel(token_ids, field_pos, header_pos, token_embed, token_pos_embed,
           field_pos_embed, header_pos_embed):
    tok = jnp.reshape(token_ids, (N,)).astype(jnp.int32)
    fld = jnp.reshape(field_pos, (N,)).astype(jnp.int32)
    hdr = jnp.reshape(header_pos, (N,)).astype(jnp.int32)
    temb = jnp.reshape(token_embed, (TB, 128))
    tpe = jnp.reshape(token_pos_embed, (MAX_LEN * EMBED,))
    fpe = jnp.reshape(field_pos_embed, (MAX_LEN * EMBED,))
    hpe = jnp.reshape(header_pos_embed, (MAX_LEN * EMBED,))
    out = _packet_embed(tok, fld, hdr, temb, tpe, fpe, hpe)
    return jnp.reshape(out, (B, L, EMBED))


# speedup vs baseline: 1.5601x; 1.5601x over previous
import functools

import jax
import jax.numpy as jnp
from jax import lax
from jax.experimental import pallas as pl
from jax.experimental.pallas import tpu as pltpu
from jax.experimental.pallas import tpu_sc as plsc

VOCAB = 1000000
MAX_LEN = 200
EMBED = 32
B = 16384
L = 50
N = B * L

NUM_CORES = 2
NUM_SUBCORES = 16
NW = NUM_CORES * NUM_SUBCORES
ROWS_W = N // NW
CHUNK = 400
PKCHUNK = CHUNK // L
NCHUNK = ROWS_W // CHUNK
SUBS = (104, 104, 104, 88)
NBUF = 4
NITER = NCHUNK // NBUF

_mesh = plsc.VectorSubcoreMesh(core_axis_name="c", subcore_axis_name="s")


@functools.partial(
    pl.kernel,
    out_type=jax.ShapeDtypeStruct((B, L, EMBED), jnp.float32),
    mesh=_mesh,
    compiler_params=pltpu.CompilerParams(needs_layout_passes=False,
                                         use_tc_tiling_on_sc=False),
    scratch_types=[
        pltpu.VMEM((NBUF, CHUNK), jnp.int32),
        pltpu.VMEM((NBUF, CHUNK), jnp.int32),
        pltpu.VMEM((NBUF, CHUNK), jnp.int32),
        pltpu.VMEM((NBUF, CHUNK, EMBED), jnp.float32),
        pltpu.VMEM((MAX_LEN, EMBED), jnp.float32),
        pltpu.VMEM((MAX_LEN, EMBED), jnp.float32),
        pltpu.VMEM((MAX_LEN, EMBED), jnp.float32),
        [pltpu.SemaphoreType.DMA] * NBUF,
        [pltpu.SemaphoreType.DMA] * NBUF,
    ],
)
def _packet_embed(tok, fld, hdr, temb, tpe, fpe, hpe, out,
                  tok_v, fld_v, hdr_v, buf_v, tpe_v, fpe_v, hpe_v,
                  gsems, osems):
    wid = lax.axis_index("s") * NUM_CORES + lax.axis_index("c")
    base_w = wid * ROWS_W
    pk_w = base_w // L

    pltpu.sync_copy(tpe, tpe_v)
    pltpu.sync_copy(fpe, fpe_v)
    pltpu.sync_copy(hpe, hpe_v)

    def stage(c, s):
        base = base_w + c * CHUNK
        pltpu.sync_copy(tok.at[pl.ds(base, CHUNK)], tok_v.at[s])
        pltpu.sync_copy(fld.at[pl.ds(base, CHUNK)], fld_v.at[s])
        pltpu.sync_copy(hdr.at[pl.ds(base, CHUNK)], hdr_v.at[s])
        off = 0
        for sub in SUBS:
            pltpu.async_copy(
                temb.at[tok_v.at[s, pl.ds(off, sub)]],
                buf_v.at[s, pl.ds(off, sub), :],
                gsems[s],
            )
            off += sub

    def drain_gathers(s):
        off = 0
        for sub in SUBS:
            pltpu.make_async_copy(
                temb.at[tok_v.at[s, pl.ds(off, sub)]],
                buf_v.at[s, pl.ds(off, sub), :],
                gsems[s],
            ).wait()
            off += sub

    def fire_out(c, s):
        pk = pk_w + c * PKCHUNK
        for p in range(PKCHUNK):
            pltpu.async_copy(buf_v.at[s, pl.ds(p * L, L), :],
                             out.at[pk + p], osems[s])

    def wait_out(c, s):
        pk = pk_w + c * PKCHUNK
        for p in range(PKCHUNK):
            pltpu.make_async_copy(buf_v.at[s, pl.ds(p * L, L), :],
                                  out.at[pk + p], osems[s]).wait()

    def compute(c, s):
        def pkt_body(p, inner):
            p50 = p * L
            fv = [fld_v[s, pl.ds(p50 + o, 16)] for o in (0, 16, 32, 34)]
            hv = [hdr_v[s, pl.ds(p50 + o, 16)] for o in (0, 16, 32, 34)]

            def lane(vecs, l):
                if l < 16:
                    return vecs[0][l]
                if l < 32:
                    return vecs[1][l - 16]
                if l < 48:
                    return vecs[2][l - 32]
                return vecs[3][l - 34]

            for l in range(L):
                f = lane(fv, l)
                h = lane(hv, l)
                r = p50 + l
                for half in (0, 16):
                    sl = pl.ds(half, 16)
                    acc = (buf_v[s, r, sl] + tpe_v[l, sl]
                           + fpe_v[f, sl] + hpe_v[h, sl])
                    buf_v[s, r, sl] = acc
            return inner

        lax.fori_loop(0, PKCHUNK, pkt_body, 0)

    stage(0, 0)

    def iter_body(k, carry):
        ii = k * NBUF
        for j in range(NBUF):
            c = ii + j
            sn = (j + 1) % NBUF

            @pl.when(jnp.logical_and(c >= NBUF - 1, c + 1 < NCHUNK))
            def _():
                wait_out(c + 1 - NBUF, sn)

            @pl.when(c + 1 < NCHUNK)
            def _():
                stage(c + 1, sn)

            drain_gathers(j)
            compute(c, j)
            fire_out(c, j)
        return carry

    lax.fori_loop(0, NITER, iter_body, 0)

    for j in range(NBUF):
        wait_out(NCHUNK - NBUF + j, j)


def kernel(token_ids, field_pos, header_pos, token_embed, token_pos_embed,
           field_pos_embed, header_pos_embed):
    tok = jnp.reshape(token_ids, (N,)).astype(jnp.int32)
    fld = jnp.reshape(field_pos, (N,)).astype(jnp.int32)
    hdr = jnp.reshape(header_pos, (N,)).astype(jnp.int32)
    return _packet_embed(tok, fld, hdr, token_embed, token_pos_embed,
                         field_pos_embed, header_pos_embed)


# speedup vs baseline: 1.5623x; 1.0014x over previous
import functools

import jax
import jax.numpy as jnp
from jax import lax
from jax.experimental import pallas as pl
from jax.experimental.pallas import tpu as pltpu
from jax.experimental.pallas import tpu_sc as plsc

VOCAB = 1000000
MAX_LEN = 200
EMBED = 32
B = 16384
L = 50
N = B * L

NUM_CORES = 2
NUM_SUBCORES = 16
NW = NUM_CORES * NUM_SUBCORES
ROWS_W = N // NW
CHUNK = 400
PKCHUNK = CHUNK // L
NCHUNK = ROWS_W // CHUNK
NBUF = 4
NITER = NCHUNK // NBUF

_mesh = plsc.VectorSubcoreMesh(core_axis_name="c", subcore_axis_name="s")


@functools.partial(
    pl.kernel,
    out_type=jax.ShapeDtypeStruct((B, L, EMBED), jnp.float32),
    mesh=_mesh,
    compiler_params=pltpu.CompilerParams(needs_layout_passes=False,
                                         use_tc_tiling_on_sc=False),
    scratch_types=[
        pltpu.VMEM((NBUF, CHUNK), jnp.int32),
        pltpu.VMEM((NBUF, CHUNK), jnp.int32),
        pltpu.VMEM((NBUF, CHUNK), jnp.int32),
        pltpu.VMEM((NBUF, CHUNK, EMBED), jnp.float32),
        pltpu.VMEM((MAX_LEN, EMBED), jnp.float32),
        pltpu.VMEM((MAX_LEN, EMBED), jnp.float32),
        pltpu.VMEM((MAX_LEN, EMBED), jnp.float32),
        [pltpu.SemaphoreType.DMA] * NBUF,
        [pltpu.SemaphoreType.DMA] * NBUF,
    ],
)
def _packet_embed(tok, fld, hdr, temb, tpe, fpe, hpe, out,
                  tok_v, fld_v, hdr_v, buf_v, tpe_v, fpe_v, hpe_v,
                  gsems, osems):
    wid = lax.axis_index("s") * NUM_CORES + lax.axis_index("c")
    base_w = wid * ROWS_W
    pk_w = base_w // L

    pltpu.sync_copy(tpe, tpe_v)
    pltpu.sync_copy(fpe, fpe_v)
    pltpu.sync_copy(hpe, hpe_v)

    def stage(c, s):
        base = base_w + c * CHUNK
        pltpu.sync_copy(tok.at[pl.ds(base, CHUNK)], tok_v.at[s])
        pltpu.sync_copy(fld.at[pl.ds(base, CHUNK)], fld_v.at[s])
        pltpu.sync_copy(hdr.at[pl.ds(base, CHUNK)], hdr_v.at[s])
        pltpu.async_copy(temb.at[tok_v.at[s]], buf_v.at[s], gsems[s])

    def drain_gather(s):
        pltpu.make_async_copy(temb.at[tok_v.at[s]], buf_v.at[s],
                              gsems[s]).wait()

    def fire_out(c, s):
        pk = pk_w + c * PKCHUNK
        for p in range(PKCHUNK):
            pltpu.async_copy(buf_v.at[s, pl.ds(p * L, L), :],
                             out.at[pk + p], osems[s])

    def wait_out(c, s):
        pk = pk_w + c * PKCHUNK
        for p in range(PKCHUNK):
            pltpu.make_async_copy(buf_v.at[s, pl.ds(p * L, L), :],
                                  out.at[pk + p], osems[s]).wait()

    def compute(c, s):
        def pkt_body(p, inner):
            p50 = p * L
            fv = [fld_v[s, pl.ds(p50 + o, 16)] for o in (0, 16, 32, 34)]
            hv = [hdr_v[s, pl.ds(p50 + o, 16)] for o in (0, 16, 32, 34)]

            def lane(vecs, l):
                if l < 16:
                    return vecs[0][l]
                if l < 32:
                    return vecs[1][l - 16]
                if l < 48:
                    return vecs[2][l - 32]
                return vecs[3][l - 34]

            for l in range(L):
                f = lane(fv, l)
                h = lane(hv, l)
                r = p50 + l
                for half in (0, 16):
                    sl = pl.ds(half, 16)
                    acc = (buf_v[s, r, sl] + tpe_v[l, sl]
                           + fpe_v[f, sl] + hpe_v[h, sl])
                    buf_v[s, r, sl] = acc
            return inner

        lax.fori_loop(0, PKCHUNK, pkt_body, 0)

    stage(0, 0)

    def iter_body(k, carry):
        ii = k * NBUF
        for j in range(NBUF):
            c = ii + j
            sn = (j + 1) % NBUF

            @pl.when(jnp.logical_and(c >= NBUF - 1, c + 1 < NCHUNK))
            def _():
                wait_out(c + 1 - NBUF, sn)

            @pl.when(c + 1 < NCHUNK)
            def _():
                stage(c + 1, sn)

            drain_gather(j)
            compute(c, j)
            fire_out(c, j)
        return carry

    lax.fori_loop(0, NITER, iter_body, 0)

    for j in range(NBUF):
        wait_out(NCHUNK - NBUF + j, j)


def kernel(token_ids, field_pos, header_pos, token_embed, token_pos_embed,
           field_pos_embed, header_pos_embed):
    tok = jnp.reshape(token_ids, (N,)).astype(jnp.int32)
    fld = jnp.reshape(field_pos, (N,)).astype(jnp.int32)
    hdr = jnp.reshape(header_pos, (N,)).astype(jnp.int32)
    return _packet_embed(tok, fld, hdr, token_embed, token_pos_embed,
                         field_pos_embed, header_pos_embed)


# speedup vs baseline: 1.5987x; 1.0233x over previous
import functools

import jax
import jax.numpy as jnp
from jax import lax
from jax.experimental import pallas as pl
from jax.experimental.pallas import tpu as pltpu
from jax.experimental.pallas import tpu_sc as plsc

VOCAB = 1000000
MAX_LEN = 200
EMBED = 32
B = 16384
L = 50
N = B * L

NUM_CORES = 2
NUM_SUBCORES = 16
NW = NUM_CORES * NUM_SUBCORES
ROWS_W = N // NW
CHUNK = 400
PKCHUNK = CHUNK // L
NCHUNK = ROWS_W // CHUNK
NBUF = 4
NITER = NCHUNK // NBUF

_mesh = plsc.VectorSubcoreMesh(core_axis_name="c", subcore_axis_name="s")


@functools.partial(
    pl.kernel,
    out_type=jax.ShapeDtypeStruct((B, L, EMBED), jnp.float32),
    mesh=_mesh,
    compiler_params=pltpu.CompilerParams(needs_layout_passes=False,
                                         use_tc_tiling_on_sc=False),
    scratch_types=[
        pltpu.VMEM((NBUF, CHUNK), jnp.int32),
        pltpu.VMEM((NBUF, CHUNK), jnp.int32),
        pltpu.VMEM((NBUF, CHUNK), jnp.int32),
        pltpu.VMEM((NBUF, CHUNK, EMBED), jnp.bfloat16),
        pltpu.VMEM((NBUF, CHUNK, EMBED), jnp.float32),
        pltpu.VMEM((MAX_LEN, EMBED), jnp.float32),
        pltpu.VMEM((MAX_LEN, EMBED), jnp.float32),
        pltpu.VMEM((MAX_LEN, EMBED), jnp.float32),
        [pltpu.SemaphoreType.DMA] * NBUF,
        [pltpu.SemaphoreType.DMA] * NBUF,
    ],
)
def _packet_embed(tok, fld, hdr, temb, tpe, fpe, hpe, out,
                  tok_v, fld_v, hdr_v, bbuf_v, obuf_v, tpe_v, fpe_v, hpe_v,
                  gsems, osems):
    wid = lax.axis_index("s") * NUM_CORES + lax.axis_index("c")
    base_w = wid * ROWS_W
    pk_w = base_w // L

    pltpu.sync_copy(tpe, tpe_v)
    pltpu.sync_copy(fpe, fpe_v)
    pltpu.sync_copy(hpe, hpe_v)

    def stage(c, s):
        base = base_w + c * CHUNK
        pltpu.sync_copy(tok.at[pl.ds(base, CHUNK)], tok_v.at[s])
        pltpu.sync_copy(fld.at[pl.ds(base, CHUNK)], fld_v.at[s])
        pltpu.sync_copy(hdr.at[pl.ds(base, CHUNK)], hdr_v.at[s])
        pltpu.async_copy(temb.at[tok_v.at[s]], bbuf_v.at[s], gsems[s])

    def drain_gather(s):
        pltpu.make_async_copy(temb.at[tok_v.at[s]], bbuf_v.at[s],
                              gsems[s]).wait()

    def fire_out(c, s):
        pk = pk_w + c * PKCHUNK
        for p in range(PKCHUNK):
            pltpu.async_copy(obuf_v.at[s, pl.ds(p * L, L), :],
                             out.at[pk + p], osems[s])

    def wait_out(c, s):
        pk = pk_w + c * PKCHUNK
        for p in range(PKCHUNK):
            pltpu.make_async_copy(
                obuf_v.at[s, pl.ds(p * L, L), :],
                out.at[pk + p], osems[s]).wait()

    iota2 = lax.iota(jnp.int32, 16) * 2

    def compute(c, s):
        def pkt_body(p, inner):
            p50 = p * L
            fv = [fld_v[s, pl.ds(p50 + o, 16)] for o in (0, 16, 32, 34)]
            hv = [hdr_v[s, pl.ds(p50 + o, 16)] for o in (0, 16, 32, 34)]

            def lane(vecs, l):
                if l < 16:
                    return vecs[0][l]
                if l < 32:
                    return vecs[1][l - 16]
                if l < 48:
                    return vecs[2][l - 32]
                return vecs[3][l - 34]

            for l in range(L):
                f = lane(fv, l)
                h = lane(hv, l)
                r = p50 + l
                w = plsc.bitcast(bbuf_v[s, r, :], jnp.int32)
                t_e = plsc.bitcast(lax.shift_left(w, 16), jnp.float32)
                t_o = plsc.bitcast(
                    lax.bitwise_and(w, jnp.int32(-65536)), jnp.float32)
                sl_e = pl.ds(0, 16)
                sl_o = pl.ds(16, 16)
                acc_e = t_e + tpe_v[l, sl_e] + fpe_v[f, sl_e] + hpe_v[h, sl_e]
                acc_o = t_o + tpe_v[l, sl_o] + fpe_v[f, sl_o] + hpe_v[h, sl_o]
                rvec = jnp.full((16,), r, jnp.int32)
                plsc.store_scatter(obuf_v.at[s], [rvec, iota2], acc_e)
                plsc.store_scatter(obuf_v.at[s], [rvec, iota2 + 1], acc_o)
            return inner

        lax.fori_loop(0, PKCHUNK, pkt_body, 0)

    stage(0, 0)

    def iter_body(k, carry):
        ii = k * NBUF
        for j in range(NBUF):
            c = ii + j
            sn = (j + 1) % NBUF

            @pl.when(jnp.logical_and(c >= NBUF - 1, c + 1 < NCHUNK))
            def _():
                wait_out(c + 1 - NBUF, sn)

            @pl.when(c + 1 < NCHUNK)
            def _():
                stage(c + 1, sn)

            drain_gather(j)
            compute(c, j)
            fire_out(c, j)
        return carry

    lax.fori_loop(0, NITER, iter_body, 0)

    for j in range(NBUF):
        wait_out(NCHUNK - NBUF + j, j)


def _split_even_odd(t):
    return jnp.concatenate([t[:, 0::2], t[:, 1::2]], axis=1)


def kernel(token_ids, field_pos, header_pos, token_embed, token_pos_embed,
           field_pos_embed, header_pos_embed):
    tok = jnp.reshape(token_ids, (N,)).astype(jnp.int32)
    fld = jnp.reshape(field_pos, (N,)).astype(jnp.int32)
    hdr = jnp.reshape(header_pos, (N,)).astype(jnp.int32)
    temb = token_embed.astype(jnp.bfloat16)
    return _packet_embed(tok, fld, hdr, temb,
                         _split_even_odd(token_pos_embed),
                         _split_even_odd(field_pos_embed),
                         _split_even_odd(header_pos_embed))
